# baseline (speedup 1.0000x reference)
import math

import jax
import jax.numpy as jnp
import numpy as np
from jax.experimental import pallas as pl

_BIG = 1e30
_BB = 8


def _sigma(i):
    return ((i & 7) << 5) | (i & 24) | ((i >> 5) & 7)


def _sw_body(x_ref, w_ref, th_ref, LD_ref, sel_ref, pif_ref, ipif_ref,
             out_ref):
    npad = w_ref.shape[0] // _BB
    m = th_ref.shape[1]
    s_tot = _BB * npad
    n = 200

    idxp = jax.lax.broadcasted_iota(jnp.int32, (s_tot, m), 0) & (npad - 1)
    il = ((idxp & 7) << 5) | (idxp & 24) | ((idxp >> 5) & 7)

    proj = jnp.dot(x_ref[...], th_ref[...],
                   preferred_element_type=jnp.float32,
                   precision=jax.lax.Precision.HIGHEST)
    key = jnp.where(il < n, proj, _BIG)
    val = jnp.broadcast_to(w_ref[...], (s_tot, m))

    k = 2
    while k <= npad:
        kp = _sigma(k) if k < npad else 0
        j = k // 2
        while j >= 1:
            jp = _sigma(j)
            if jp >= 8:
                g = s_tot // (2 * jp)
                k4 = key.reshape(g, 2, jp, m)
                v4 = val.reshape(g, 2, jp, m)
                ka, kb = k4[:, 0], k4[:, 1]
                va, vb = v4[:, 0], v4[:, 1]
                le = ka <= kb
                if kp:
                    i4 = idxp.reshape(g, 2, jp, m)
                    desc = (i4[:, 0] & kp) != 0
                    sel = jnp.logical_xor(le, desc)
                else:
                    sel = le
                k0 = jnp.where(sel, ka, kb)
                k1 = jnp.where(sel, kb, ka)
                v0 = jnp.where(sel, va, vb)
                v1 = jnp.where(sel, vb, va)
                key = jnp.stack([k0, k1], axis=1).reshape(s_tot, m)
                val = jnp.stack([v0, v1], axis=1).reshape(s_tot, m)
            else:
                bitset = (idxp & jp) != 0
                pk = jnp.where(bitset, jnp.roll(key, jp, axis=0),
                               jnp.roll(key, -jp, axis=0))
                pv = jnp.where(bitset, jnp.roll(val, jp, axis=0),
                               jnp.roll(val, -jp, axis=0))
                le = key <= pk
                ge = ~(key < pk)
                if kp:
                    asc = (idxp & kp) == 0
                    want_min = jnp.logical_xor(bitset, asc)
                else:
                    want_min = ~bitset
                take_self = jnp.logical_or(
                    jnp.logical_and(want_min, le),
                    jnp.logical_and(~want_min, ge))
                key = jnp.where(take_self, key, pk)
                val = jnp.where(take_self, val, pv)
            j //= 2
        k *= 2

    LD = LD_ref[...]
    L, D = LD[:npad], LD[npad:]
    key_m = jnp.where(il < n, key, 0.0)
    cs, ts = [], []
    for bb in range(_BB):
        sl = slice(bb * npad, (bb + 1) * npad)
        cs.append(jnp.dot(L, val[sl], preferred_element_type=jnp.float32,
                          precision=jax.lax.Precision.HIGHEST))
        ts.append(jnp.dot(D, key_m[sl], preferred_element_type=jnp.float32,
                          precision=jax.lax.Precision.HIGHEST))
    c = jnp.concatenate(cs, axis=0)
    t = jnp.concatenate(ts, axis=0)

    prod = t * jnp.sin(c * pif_ref[...])
    acc = jnp.dot(sel_ref[...], prod, preferred_element_type=jnp.float32,
                  precision=jax.lax.Precision.HIGHEST)
    out_ref[0] = math.sqrt(2.0) * acc * ipif_ref[...]


def kernel(X, W, theta, freqs):
    b, n, d_in = X.shape
    m = theta.shape[0]
    npad = 1 << (n - 1).bit_length()
    assert npad == 256 and n == 200, "specialized to n=200 (npad=256)"
    s_tot = _BB * npad

    ilv = _sigma(np.arange(npad))
    src = np.minimum(ilv, n - 1)
    gidx = (np.arange(b)[:, None] * n + src[None, :]).reshape(-1)
    x2 = jnp.take(X.reshape(b * n, d_in), jnp.asarray(gidx), axis=0)

    wn = W / jnp.sum(W, axis=-1, keepdims=True)
    wpad = jnp.pad(wn, ((0, 0), (0, npad - n)))
    wcol = wpad[:, jnp.asarray(ilv)].reshape(b * npad, 1)

    th_t = theta.T
    Lm = (ilv[None, :] <= ilv[:, None]).astype(np.float32)
    Dm = np.eye(npad, dtype=np.float32) - \
        (ilv[None, :] == ilv[:, None] + 1).astype(np.float32)
    LD = jnp.asarray(np.concatenate([Lm, Dm], axis=0))

    seg = jnp.arange(s_tot, dtype=jnp.int32) // npad
    sel_m = (seg[None, :] == jnp.arange(_BB, dtype=jnp.int32)[:, None])
    sel_m = sel_m.astype(jnp.float32)
    pif = (jnp.pi * freqs)[None, :].astype(jnp.float32)
    ipif = (1.0 / (jnp.pi * freqs))[None, :].astype(jnp.float32)

    grid = b // _BB
    out = pl.pallas_call(
        _sw_body,
        grid=(grid,),
        in_specs=[
            pl.BlockSpec((s_tot, d_in), lambda i: (i, 0)),
            pl.BlockSpec((s_tot, 1), lambda i: (i, 0)),
            pl.BlockSpec((d_in, m), lambda i: (0, 0)),
            pl.BlockSpec((2 * npad, npad), lambda i: (0, 0)),
            pl.BlockSpec((_BB, s_tot), lambda i: (0, 0)),
            pl.BlockSpec((1, m), lambda i: (0, 0)),
            pl.BlockSpec((1, m), lambda i: (0, 0)),
        ],
        out_specs=pl.BlockSpec((1, _BB, m), lambda i: (i, 0, 0)),
        out_shape=jax.ShapeDtypeStruct((grid, _BB, m), jnp.float32),
    )(x2, wcol, th_t, LD, sel_m, pif, ipif)
    return out.reshape(b, m)


# speedup vs baseline: 1.0145x; 1.0145x over previous
import math

import jax
import jax.numpy as jnp
import numpy as np
from jax.experimental import pallas as pl

_BIG = 1e30
_BB = 4


def _sigma(i):
    return ((i & 7) << 5) | (i & 24) | ((i >> 5) & 7)


def _sw_body(x_ref, w_ref, th_ref, LD_ref, sel_ref, pif_ref, ipif_ref,
             out_ref):
    npad = w_ref.shape[0] // _BB
    m = th_ref.shape[1]
    s_tot = _BB * npad
    n = 200

    idxp = jax.lax.broadcasted_iota(jnp.int32, (s_tot, m), 0) & (npad - 1)
    il = ((idxp & 7) << 5) | (idxp & 24) | ((idxp >> 5) & 7)

    proj = jnp.dot(x_ref[...], th_ref[...],
                   preferred_element_type=jnp.float32,
                   precision=jax.lax.Precision.HIGHEST)
    key = jnp.where(il < n, proj, _BIG)
    val = jnp.broadcast_to(w_ref[...], (s_tot, m))

    k = 2
    while k <= npad:
        kp = _sigma(k) if k < npad else 0
        j = k // 2
        while j >= 1:
            jp = _sigma(j)
            if jp >= 8:
                g = s_tot // (2 * jp)
                k4 = key.reshape(g, 2, jp, m)
                v4 = val.reshape(g, 2, jp, m)
                ka, kb = k4[:, 0], k4[:, 1]
                va, vb = v4[:, 0], v4[:, 1]
                le = ka <= kb
                if kp:
                    i4 = idxp.reshape(g, 2, jp, m)
                    desc = (i4[:, 0] & kp) != 0
                    sel = jnp.logical_xor(le, desc)
                else:
                    sel = le
                k0 = jnp.where(sel, ka, kb)
                k1 = jnp.where(sel, kb, ka)
                v0 = jnp.where(sel, va, vb)
                v1 = jnp.where(sel, vb, va)
                key = jnp.stack([k0, k1], axis=1).reshape(s_tot, m)
                val = jnp.stack([v0, v1], axis=1).reshape(s_tot, m)
            else:
                bitset = (idxp & jp) != 0
                pk = jnp.where(bitset, jnp.roll(key, jp, axis=0),
                               jnp.roll(key, -jp, axis=0))
                pv = jnp.where(bitset, jnp.roll(val, jp, axis=0),
                               jnp.roll(val, -jp, axis=0))
                le = key <= pk
                ge = ~(key < pk)
                if kp:
                    asc = (idxp & kp) == 0
                    want_min = jnp.logical_xor(bitset, asc)
                else:
                    want_min = ~bitset
                take_self = jnp.logical_or(
                    jnp.logical_and(want_min, le),
                    jnp.logical_and(~want_min, ge))
                key = jnp.where(take_self, key, pk)
                val = jnp.where(take_self, val, pv)
            j //= 2
        k *= 2

    LD = LD_ref[...]
    L, D = LD[:npad], LD[npad:]
    key_m = jnp.where(il < n, key, 0.0)
    cs, ts = [], []
    for bb in range(_BB):
        sl = slice(bb * npad, (bb + 1) * npad)
        cs.append(jnp.dot(L, val[sl], preferred_element_type=jnp.float32,
                          precision=jax.lax.Precision.HIGHEST))
        ts.append(jnp.dot(D, key_m[sl], preferred_element_type=jnp.float32,
                          precision=jax.lax.Precision.HIGHEST))
    c = jnp.concatenate(cs, axis=0)
    t = jnp.concatenate(ts, axis=0)

    prod = t * jnp.sin(c * pif_ref[...])
    acc = jnp.dot(sel_ref[...], prod, preferred_element_type=jnp.float32,
                  precision=jax.lax.Precision.HIGHEST)
    out_ref[0] = math.sqrt(2.0) * acc * ipif_ref[...]


def kernel(X, W, theta, freqs):
    b, n, d_in = X.shape
    m = theta.shape[0]
    npad = 1 << (n - 1).bit_length()
    assert npad == 256 and n == 200, "specialized to n=200 (npad=256)"
    s_tot = _BB * npad

    ilv = _sigma(np.arange(npad))
    src = np.minimum(ilv, n - 1)
    gidx = (np.arange(b)[:, None] * n + src[None, :]).reshape(-1)
    x2 = jnp.take(X.reshape(b * n, d_in), jnp.asarray(gidx), axis=0)

    wn = W / jnp.sum(W, axis=-1, keepdims=True)
    wpad = jnp.pad(wn, ((0, 0), (0, npad - n)))
    wcol = wpad[:, jnp.asarray(ilv)].reshape(b * npad, 1)

    th_t = theta.T
    Lm = (ilv[None, :] <= ilv[:, None]).astype(np.float32)
    Dm = np.eye(npad, dtype=np.float32) - \
        (ilv[None, :] == ilv[:, None] + 1).astype(np.float32)
    LD = jnp.asarray(np.concatenate([Lm, Dm], axis=0))

    seg = jnp.arange(s_tot, dtype=jnp.int32) // npad
    sel_m = (seg[None, :] == jnp.arange(_BB, dtype=jnp.int32)[:, None])
    sel_m = sel_m.astype(jnp.float32)
    pif = (jnp.pi * freqs)[None, :].astype(jnp.float32)
    ipif = (1.0 / (jnp.pi * freqs))[None, :].astype(jnp.float32)

    grid = b // _BB
    out = pl.pallas_call(
        _sw_body,
        grid=(grid,),
        in_specs=[
            pl.BlockSpec((s_tot, d_in), lambda i: (i, 0)),
            pl.BlockSpec((s_tot, 1), lambda i: (i, 0)),
            pl.BlockSpec((d_in, m), lambda i: (0, 0)),
            pl.BlockSpec((2 * npad, npad), lambda i: (0, 0)),
            pl.BlockSpec((_BB, s_tot), lambda i: (0, 0)),
            pl.BlockSpec((1, m), lambda i: (0, 0)),
            pl.BlockSpec((1, m), lambda i: (0, 0)),
        ],
        out_specs=pl.BlockSpec((1, _BB, m), lambda i: (i, 0, 0)),
        out_shape=jax.ShapeDtypeStruct((grid, _BB, m), jnp.float32),
    )(x2, wcol, th_t, LD, sel_m, pif, ipif)
    return out.reshape(b, m)


# speedup vs baseline: 1.1009x; 1.0853x over previous
import math

import jax
import jax.numpy as jnp
import numpy as np
from jax.experimental import pallas as pl

_BIG = 1e30
_BB = 4


def _sigma(i):
    return ((i & 7) << 5) | (i & 24) | ((i >> 5) & 7)


def _sw_body(x_ref, w_ref, th_ref, LD_ref, sel_ref, pif_ref, ipif_ref,
             out_ref):
    n = x_ref.shape[0] // _BB
    npad = w_ref.shape[0] // _BB
    m = th_ref.shape[1]
    s_tot = _BB * npad

    idxp = jax.lax.broadcasted_iota(jnp.int32, (s_tot, m), 0) & (npad - 1)
    il = ((idxp & 7) << 5) | (idxp & 24) | ((idxp >> 5) & 7)

    proj = jnp.dot(x_ref[...], th_ref[...],
                   preferred_element_type=jnp.float32,
                   precision=jax.lax.Precision.HIGHEST)
    pad = jnp.full((npad - n, m), _BIG, jnp.float32)
    pieces = []
    for bb in range(_BB):
        pieces.append(proj[bb * n:(bb + 1) * n])
        pieces.append(pad)
    key = jnp.concatenate(pieces, axis=0)
    val = jnp.broadcast_to(w_ref[...], (s_tot, m))

    k = 2
    while k <= npad:
        kp = _sigma(k) if k < npad else 0
        j = k // 2
        while j >= 1:
            jp = _sigma(j)
            if jp >= 8:
                g = s_tot // (2 * jp)
                k4 = key.reshape(g, 2, jp, m)
                v4 = val.reshape(g, 2, jp, m)
                ka, kb = k4[:, 0], k4[:, 1]
                va, vb = v4[:, 0], v4[:, 1]
                le = ka <= kb
                if kp:
                    i4 = idxp.reshape(g, 2, jp, m)
                    desc = (i4[:, 0] & kp) != 0
                    sel = jnp.logical_xor(le, desc)
                else:
                    sel = le
                k0 = jnp.where(sel, ka, kb)
                k1 = jnp.where(sel, kb, ka)
                v0 = jnp.where(sel, va, vb)
                v1 = jnp.where(sel, vb, va)
                key = jnp.stack([k0, k1], axis=1).reshape(s_tot, m)
                val = jnp.stack([v0, v1], axis=1).reshape(s_tot, m)
            else:
                bitset = (idxp & jp) != 0
                pk = jnp.where(bitset, jnp.roll(key, jp, axis=0),
                               jnp.roll(key, -jp, axis=0))
                pv = jnp.where(bitset, jnp.roll(val, jp, axis=0),
                               jnp.roll(val, -jp, axis=0))
                le = key <= pk
                ge = ~(key < pk)
                if kp:
                    asc = (idxp & kp) == 0
                    want_min = jnp.logical_xor(bitset, asc)
                else:
                    want_min = ~bitset
                take_self = jnp.logical_or(
                    jnp.logical_and(want_min, le),
                    jnp.logical_and(~want_min, ge))
                key = jnp.where(take_self, key, pk)
                val = jnp.where(take_self, val, pv)
            j //= 2
        k *= 2

    LD = LD_ref[...]
    L, D = LD[:npad], LD[npad:]
    key_m = jnp.where(il < n, key, 0.0)
    cs, ts = [], []
    for bb in range(_BB):
        sl = slice(bb * npad, (bb + 1) * npad)
        cs.append(jnp.dot(L, val[sl], preferred_element_type=jnp.float32,
                          precision=jax.lax.Precision.HIGHEST))
        ts.append(jnp.dot(D, key_m[sl], preferred_element_type=jnp.float32,
                          precision=jax.lax.Precision.HIGHEST))
    c = jnp.concatenate(cs, axis=0)
    t = jnp.concatenate(ts, axis=0)

    prod = t * jnp.sin(c * pif_ref[...])
    acc = jnp.dot(sel_ref[...], prod, preferred_element_type=jnp.float32,
                  precision=jax.lax.Precision.HIGHEST)
    out_ref[0] = math.sqrt(2.0) * acc * ipif_ref[...]


def kernel(X, W, theta, freqs):
    b, n, d_in = X.shape
    m = theta.shape[0]
    npad = 1 << (n - 1).bit_length()
    assert npad == 256 and n == 200, "specialized to n=200 (npad=256)"
    s_tot = _BB * npad

    ilv = _sigma(np.arange(npad))
    x2 = X.reshape(b * n, d_in)

    wn = W / jnp.sum(W, axis=-1, keepdims=True)
    wcol = jnp.pad(wn, ((0, 0), (0, npad - n))).reshape(b * npad, 1)

    th_t = theta.T
    Lm = (ilv[None, :] <= ilv[:, None]).astype(np.float32)
    Dm = np.eye(npad, dtype=np.float32) - \
        (ilv[None, :] == ilv[:, None] + 1).astype(np.float32)
    LD = jnp.asarray(np.concatenate([Lm, Dm], axis=0))

    seg = jnp.arange(s_tot, dtype=jnp.int32) // npad
    sel_m = (seg[None, :] == jnp.arange(_BB, dtype=jnp.int32)[:, None])
    sel_m = sel_m.astype(jnp.float32)
    pif = (jnp.pi * freqs)[None, :].astype(jnp.float32)
    ipif = (1.0 / (jnp.pi * freqs))[None, :].astype(jnp.float32)

    grid = b // _BB
    out = pl.pallas_call(
        _sw_body,
        grid=(grid,),
        in_specs=[
            pl.BlockSpec((_BB * n, d_in), lambda i: (i, 0)),
            pl.BlockSpec((s_tot, 1), lambda i: (i, 0)),
            pl.BlockSpec((d_in, m), lambda i: (0, 0)),
            pl.BlockSpec((2 * npad, npad), lambda i: (0, 0)),
            pl.BlockSpec((_BB, s_tot), lambda i: (0, 0)),
            pl.BlockSpec((1, m), lambda i: (0, 0)),
            pl.BlockSpec((1, m), lambda i: (0, 0)),
        ],
        out_specs=pl.BlockSpec((1, _BB, m), lambda i: (i, 0, 0)),
        out_shape=jax.ShapeDtypeStruct((grid, _BB, m), jnp.float32),
    )(x2, wcol, th_t, LD, sel_m, pif, ipif)
    return out.reshape(b, m)


# speedup vs baseline: 1.1968x; 1.0871x over previous
import math

import jax
import jax.numpy as jnp
import numpy as np
from jax.experimental import pallas as pl

_BIG = 1e30
_BB = 4


def _sigma(i):
    return ((i & 7) << 5) | (i & 24) | ((i >> 5) & 7)


def _sw_body(x_ref, w_ref, th_ref, LD_ref, sel_ref, fh_ref, ipif_ref,
             out_ref):
    n = x_ref.shape[0] // _BB
    npad = w_ref.shape[0] // _BB
    m = th_ref.shape[1]
    s_tot = _BB * npad

    idxp = jax.lax.broadcasted_iota(jnp.int32, (s_tot, m), 0) & (npad - 1)
    il = ((idxp & 7) << 5) | (idxp & 24) | ((idxp >> 5) & 7)

    proj = jnp.dot(x_ref[...], th_ref[...],
                   preferred_element_type=jnp.float32,
                   precision=jax.lax.Precision.HIGHEST)
    pad = jnp.full((npad - n, m), _BIG, jnp.float32)
    pieces = []
    for bb in range(_BB):
        pieces.append(proj[bb * n:(bb + 1) * n])
        pieces.append(pad)
    key = jnp.concatenate(pieces, axis=0)
    val = jnp.broadcast_to(w_ref[...], (s_tot, m))

    k = 2
    while k <= npad:
        kp = _sigma(k) if k < npad else 0
        j = k // 2
        while j >= 1:
            jp = _sigma(j)
            if jp >= 8:
                g = s_tot // (2 * jp)
                k4 = key.reshape(g, 2, jp, m)
                v4 = val.reshape(g, 2, jp, m)
                ka, kb = k4[:, 0], k4[:, 1]
                va, vb = v4[:, 0], v4[:, 1]
                le = ka <= kb
                if kp:
                    i4 = idxp.reshape(g, 2, jp, m)
                    desc = (i4[:, 0] & kp) != 0
                    sel = jnp.logical_xor(le, desc)
                else:
                    sel = le
                k0 = jnp.where(sel, ka, kb)
                k1 = jnp.where(sel, kb, ka)
                v0 = jnp.where(sel, va, vb)
                v1 = jnp.where(sel, vb, va)
                key = jnp.stack([k0, k1], axis=1).reshape(s_tot, m)
                val = jnp.stack([v0, v1], axis=1).reshape(s_tot, m)
            else:
                bitset = (idxp & jp) != 0
                pk = jnp.where(bitset, jnp.roll(key, jp, axis=0),
                               jnp.roll(key, -jp, axis=0))
                pv = jnp.where(bitset, jnp.roll(val, jp, axis=0),
                               jnp.roll(val, -jp, axis=0))
                le = key <= pk
                ge = ~(key < pk)
                if kp:
                    asc = (idxp & kp) == 0
                    want_min = jnp.logical_xor(bitset, asc)
                else:
                    want_min = ~bitset
                take_self = jnp.logical_or(
                    jnp.logical_and(want_min, le),
                    jnp.logical_and(~want_min, ge))
                key = jnp.where(take_self, key, pk)
                val = jnp.where(take_self, val, pv)
            j //= 2
        k *= 2

    LD = LD_ref[...]
    L, D = LD[:npad], LD[npad:]
    key_m = jnp.where(il < n, key, 0.0)
    cs, ts = [], []
    for bb in range(_BB):
        sl = slice(bb * npad, (bb + 1) * npad)
        cs.append(jnp.dot(L, val[sl], preferred_element_type=jnp.float32,
                          precision=jax.lax.Precision.HIGHEST))
        ts.append(jnp.dot(D, key_m[sl], preferred_element_type=jnp.float32,
                          precision=jax.lax.Precision.HIGHEST))
    c = jnp.concatenate(cs, axis=0)
    t = jnp.concatenate(ts, axis=0)

    h = c * fh_ref[...]
    v = h - jnp.floor(h)
    w2 = v + v
    rv = jnp.floor(w2 + 0.5)
    w = w2 - rv
    w = jnp.where(rv == 1.0, -w, w)
    z = w * w
    s = w * (3.14159265358979 + z * (-5.16771278004997 + z * (
        2.55016403987735 + z * (-0.59926452932079 + z * (
            0.08214588661113 + z * -0.00737043094571)))))
    prod = t * s
    acc = jnp.dot(sel_ref[...], prod, preferred_element_type=jnp.float32,
                  precision=jax.lax.Precision.HIGHEST)
    out_ref[0] = math.sqrt(2.0) * acc * ipif_ref[...]


def kernel(X, W, theta, freqs):
    b, n, d_in = X.shape
    m = theta.shape[0]
    npad = 1 << (n - 1).bit_length()
    assert npad == 256 and n == 200, "specialized to n=200 (npad=256)"
    s_tot = _BB * npad

    ilv = _sigma(np.arange(npad))
    x2 = X.reshape(b * n, d_in)

    wn = W / jnp.sum(W, axis=-1, keepdims=True)
    wcol = jnp.pad(wn, ((0, 0), (0, npad - n))).reshape(b * npad, 1)

    th_t = theta.T
    Lm = (ilv[None, :] <= ilv[:, None]).astype(np.float32)
    Dm = np.eye(npad, dtype=np.float32) - \
        (ilv[None, :] == ilv[:, None] + 1).astype(np.float32)
    LD = jnp.asarray(np.concatenate([Lm, Dm], axis=0))

    seg = jnp.arange(s_tot, dtype=jnp.int32) // npad
    sel_m = (seg[None, :] == jnp.arange(_BB, dtype=jnp.int32)[:, None])
    sel_m = sel_m.astype(jnp.float32)
    fh = (0.5 * freqs)[None, :].astype(jnp.float32)
    ipif = (1.0 / (jnp.pi * freqs))[None, :].astype(jnp.float32)

    grid = b // _BB
    out = pl.pallas_call(
        _sw_body,
        grid=(grid,),
        in_specs=[
            pl.BlockSpec((_BB * n, d_in), lambda i: (i, 0)),
            pl.BlockSpec((s_tot, 1), lambda i: (i, 0)),
            pl.BlockSpec((d_in, m), lambda i: (0, 0)),
            pl.BlockSpec((2 * npad, npad), lambda i: (0, 0)),
            pl.BlockSpec((_BB, s_tot), lambda i: (0, 0)),
            pl.BlockSpec((1, m), lambda i: (0, 0)),
            pl.BlockSpec((1, m), lambda i: (0, 0)),
        ],
        out_specs=pl.BlockSpec((1, _BB, m), lambda i: (i, 0, 0)),
        out_shape=jax.ShapeDtypeStruct((grid, _BB, m), jnp.float32),
    )(x2, wcol, th_t, LD, sel_m, fh, ipif)
    return out.reshape(b, m)
